# baseline (speedup 1.0000x reference)
import functools
import jax
import jax.numpy as jnp
from jax import lax
from jax.experimental import pallas as pl
from jax.experimental.pallas import tpu as pltpu, tpu_sc as plsc

N = 4096
E = 131072
D = 128
H = 64
C = 16

NC, NS, L = 2, 16, 16
NW = NC * NS

CH_ROWS = 256
CH_W = CH_ROWS * N
N_SWEEPS = N // (CH_ROWS * NC)
DUMMY_B = CH_W
DUMMY_C = N

_mesh = plsc.VectorSubcoreMesh(core_axis_name="c", subcore_axis_name="s")
_sc_params = pltpu.CompilerParams(needs_layout_passes=False)


def _sigmoid(t):
    return 1.0 / (1.0 + jnp.exp(-t))


def _tc_pre_body(emb_ref, x_ref, w1a_ref, w1b_ref, w1c_ref, b1_ref,
                 embn_ref, wm1_ref, p_ref, xw_ref):
    kv = jnp.dot(embn_ref[...], w1c_ref[...],
                 preferred_element_type=jnp.float32, precision=lax.Precision.HIGHEST) + b1_ref[...]
    p = jnp.dot(emb_ref[...], w1a_ref[...],
                preferred_element_type=jnp.float32, precision=lax.Precision.HIGHEST) + kv
    q = jnp.dot(emb_ref[...], w1b_ref[...],
                preferred_element_type=jnp.float32, precision=lax.Precision.HIGHEST)
    p_ref[...] = jnp.concatenate([p, q], axis=1)
    xw_ref[...] = jnp.dot(x_ref[...], wm1_ref[...],
                          preferred_element_type=jnp.float32, precision=lax.Precision.HIGHEST)


def _tc_pre(embed, x, w1a, w1b, w1c, b1, embn, wm1):
    return pl.pallas_call(
        _tc_pre_body,
        out_shape=[
            jax.ShapeDtypeStruct((N, 2 * H), jnp.float32),
            jax.ShapeDtypeStruct((N, D), jnp.float32),
        ],
    )(embed, x, w1a, w1b, w1c, b1, embn, wm1)


EW_A = E // NW
KA = 256


def _sc_a_body(pq_hbm, row_hbm, col_hbm, w2b_hbm, b2b_hbm, vals_hbm,
               rbuf, cbuf, pg, qg, vbuf, w2b, b2b, sem, sem2):
    cid = lax.axis_index("c")
    sid = lax.axis_index("s")
    wid = sid * NC + cid
    pltpu.sync_copy(w2b_hbm, w2b)
    pltpu.sync_copy(b2b_hbm, b2b)

    def chunk(ch, _):
        base = wid * EW_A + ch * KA
        pltpu.sync_copy(row_hbm.at[pl.ds(base, KA)], rbuf)
        pltpu.sync_copy(col_hbm.at[pl.ds(base, KA)], cbuf)
        cps = []
        for j in range(KA // 128):
            cps.append(pltpu.async_copy(
                pq_hbm.at[rbuf.at[pl.ds(j * 128, 128)]],
                pg.at[pl.ds(j * 128, 128)], sem))
            cps.append(pltpu.async_copy(
                pq_hbm.at[cbuf.at[pl.ds(j * 128, 128)]],
                qg.at[pl.ds(j * 128, 128)], sem2))
        for cp in cps:
            cp.wait()

        lanes = lax.iota(jnp.int32, L)
        b2 = b2b[pl.ds(0, L)]

        def block(b, _):
            ev = b * L + lanes
            acc = jnp.zeros((L,), jnp.float32)
            for d in range(H):
                dv = jnp.full((L,), d, jnp.int32)
                pv = plsc.load_gather(pg, [ev, dv])
                qv = plsc.load_gather(qg, [ev, dv + H])
                hv = jnp.maximum(pv + qv, 0.0)
                acc = acc + hv * w2b[d, pl.ds(0, L)]
            vbuf[pl.ds(b * L, L)] = _sigmoid(acc + b2)
            return _
        lax.fori_loop(0, KA // L, block, 0)
        pltpu.sync_copy(vbuf, vals_hbm.at[pl.ds(base, KA)])
        return _
    lax.fori_loop(0, EW_A // KA, chunk, 0)


def _sc_a(pq, edge_row, edge_col, w2b, b2b):
    return pl.kernel(
        _sc_a_body,
        out_type=jax.ShapeDtypeStruct((E,), jnp.float32),
        mesh=_mesh,
        compiler_params=_sc_params,
        scratch_types=[
            pltpu.VMEM((KA,), jnp.int32),
            pltpu.VMEM((KA,), jnp.int32),
            pltpu.VMEM((KA, 2 * H), jnp.float32),
            pltpu.VMEM((KA, 2 * H), jnp.float32),
            pltpu.VMEM((KA,), jnp.float32),
            pltpu.VMEM((H, L), jnp.float32),
            pltpu.VMEM((L,), jnp.float32),
            pltpu.SemaphoreType.DMA,
            pltpu.SemaphoreType.DMA,
        ],
    )(pq, edge_row, edge_col, w2b, b2b)


EW_B = E // NS
KB = 1024


def _sc_b_body(row_hbm, col_hbm, vals_hbm, m_hbm,
               rbuf, cbuf, vbuf, iidx, zbuf, acc, sem, semz):
    cid = lax.axis_index("c")
    sid = lax.axis_index("s")
    zv = jnp.zeros((L,), jnp.float32)

    def zb(i, _):
        zbuf[pl.ds(i * L, L)] = zv
        return _
    lax.fori_loop(0, 16384 // L, zb, 0)
    lanes = lax.iota(jnp.int32, L)

    def sweep(s, _):
        base_row = s * (CH_ROWS * NC) + cid * CH_ROWS
        for j in range(4):
            pltpu.sync_copy(zbuf, acc.at[pl.ds(sid * 65536 + j * 16384, 16384)])
        plsc.subcore_barrier()

        def chunk(ch, _):
            ebase = sid * EW_B + ch * KB
            pltpu.sync_copy(row_hbm.at[pl.ds(ebase, KB)], rbuf)
            pltpu.sync_copy(col_hbm.at[pl.ds(ebase, KB)], cbuf)
            pltpu.sync_copy(vals_hbm.at[pl.ds(ebase, KB)], vbuf)

            def block(b, _):
                r16 = rbuf[pl.ds(b * L, L)]
                c16 = cbuf[pl.ds(b * L, L)]
                lidx = (r16 - base_row) * N + c16
                m = (r16 >= base_row) & (r16 < base_row + CH_ROWS)
                lidx = jnp.where(m, lidx, DUMMY_B)
                iidx[b // 8, pl.ds((b % 8) * L, L)] = lidx
                return _
            lax.fori_loop(0, KB // L, block, 0)
            cps = []
            for j in range(KB // 128):
                cps.append(pltpu.async_copy(
                    vbuf.at[pl.ds(j * 128, 128)],
                    acc.at[iidx.at[j]], sem, add=True))
            for cp in cps:
                cp.wait()
            return _
        lax.fori_loop(0, EW_B // KB, chunk, 0)
        plsc.subcore_barrier()
        pltpu.sync_copy(
            acc.at[pl.ds(sid * 65536, 65536)],
            m_hbm.at[pl.ds(base_row * N + sid * 65536, 65536)])
        plsc.subcore_barrier()
        return _
    lax.fori_loop(0, N_SWEEPS, sweep, 0)


def _sc_b(edge_row, edge_col, vals):
    return pl.kernel(
        _sc_b_body,
        out_type=jax.ShapeDtypeStruct((N * N,), jnp.float32),
        mesh=_mesh,
        compiler_params=_sc_params,
        scratch_types=[
            pltpu.VMEM((KB,), jnp.int32),
            pltpu.VMEM((KB,), jnp.int32),
            pltpu.VMEM((KB,), jnp.float32),
            pltpu.VMEM((KB // 128, 128), jnp.int32),
            pltpu.VMEM((16384,), jnp.float32),
            pltpu.VMEM_SHARED((CH_W + 64,), jnp.float32),
            pltpu.SemaphoreType.DMA,
            pltpu.SemaphoreType.DMA,
        ],
    )(edge_row, edge_col, vals)


EW_C = E // NW
KC = 128


def _sc_c_body(row_hbm, col_hbm, ev_hbm, m_hbm, xw_hbm, nid_hbm,
               h1p_hbm, cp_hbm,
               rbuf, cbuf, evbuf, fidx, ridx, cidx, fw, rw, wbuf,
               xwg, scb, zbuf2, nbuf, hacc, cacc,
               sem, sem2, sem3):
    cid = lax.axis_index("c")
    sid = lax.axis_index("s")
    zv = jnp.zeros((L,), jnp.float32)

    def zb(i, _):
        zbuf2[i // 8, pl.ds((i % 8) * L, L)] = zv
        return _
    lax.fori_loop(0, 128 * D // L, zb, 0)
    for j in range(2):
        pltpu.sync_copy(zbuf2, hacc.at[pl.ds(sid * 256 + j * 128, 128)])

    @pl.when(sid == 0)
    def _():
        for j in range(N // 128):
            pltpu.sync_copy(zbuf2.at[0], cacc.at[pl.ds(j * 128, 128)])
    pltpu.sync_copy(nid_hbm, nbuf)
    plsc.subcore_barrier()

    lanes = lax.iota(jnp.int32, L)

    def chunk(ch, _):
        gwid = cid * NS + sid
        ebase = gwid * EW_C + ch * KC
        pltpu.sync_copy(row_hbm.at[pl.ds(ebase, KC)], rbuf)
        pltpu.sync_copy(col_hbm.at[pl.ds(ebase, KC)], cbuf)
        pltpu.sync_copy(ev_hbm.at[pl.ds(ebase, KC)], evbuf)
        nid = nbuf[pl.ds(0, L)]

        def block(b, _):
            r16 = rbuf[pl.ds(b * L, L)]
            c16 = cbuf[pl.ds(b * L, L)]
            fidx[pl.ds(b * L, L)] = r16 * N + c16
            ridx[pl.ds(b * L, L)] = c16 * N + r16
            cidx[pl.ds(b * L, L)] = jnp.where(r16 == nid, c16, DUMMY_C)
            return _
        lax.fori_loop(0, KC // L, block, 0)
        cpf = pltpu.async_copy(m_hbm.at[fidx], fw, sem)
        cpr = pltpu.async_copy(m_hbm.at[ridx], rw, sem2)
        cpx = pltpu.async_copy(xw_hbm.at[cbuf], xwg, sem3)
        cpf.wait()
        cpr.wait()
        cpx.wait()

        def wblock(b, _):
            r16 = rbuf[pl.ds(b * L, L)]
            c16 = cbuf[pl.ds(b * L, L)]
            e16 = evbuf[pl.ds(b * L, L)]
            f16 = fw[pl.ds(b * L, L)]
            v16 = rw[pl.ds(b * L, L)]
            w = e16 * 0.5 * (f16 + v16)
            w = jnp.where(r16 == c16, 0.0, w)
            wbuf[pl.ds(b * L, L)] = w
            return _
        lax.fori_loop(0, KC // L, wblock, 0)

        def scale(eb, _):
            ev = eb * L + lanes
            w16 = wbuf[pl.ds(eb * L, L)]
            for d in range(D):
                dv = jnp.full((L,), d, jnp.int32)
                v = plsc.load_gather(xwg, [ev, dv])
                plsc.store_scatter(scb, [ev, dv], v * w16)
            return _
        lax.fori_loop(0, KC // L, scale, 0)

        pltpu.sync_copy(scb, hacc.at[rbuf], add=True)
        pltpu.sync_copy(wbuf, cacc.at[cidx], add=True)
        return _
    lax.fori_loop(0, EW_C // KC, chunk, 0)
    plsc.subcore_barrier()
    for j in range(2):
        pltpu.sync_copy(hacc.at[pl.ds(sid * 256 + j * 128, 128)],
                        h1p_hbm.at[cid, pl.ds(sid * 256 + j * 128, 128)])

    @pl.when(sid == 0)
    def _():
        pltpu.sync_copy(cacc.at[pl.ds(0, N)], cp_hbm.at[cid])


def _sc_c(edge_row, edge_col, edge_vals, m, xw, nid_vec):
    return pl.kernel(
        _sc_c_body,
        out_type=(
            jax.ShapeDtypeStruct((NC, N, D), jnp.float32),
            jax.ShapeDtypeStruct((NC, N), jnp.float32),
        ),
        mesh=_mesh,
        compiler_params=_sc_params,
        scratch_types=[
            pltpu.VMEM((KC,), jnp.int32),
            pltpu.VMEM((KC,), jnp.int32),
            pltpu.VMEM((KC,), jnp.float32),
            pltpu.VMEM((KC,), jnp.int32),
            pltpu.VMEM((KC,), jnp.int32),
            pltpu.VMEM((KC,), jnp.int32),
            pltpu.VMEM((KC,), jnp.float32),
            pltpu.VMEM((KC,), jnp.float32),
            pltpu.VMEM((KC,), jnp.float32),
            pltpu.VMEM((KC, D), jnp.float32),
            pltpu.VMEM((KC, D), jnp.float32),
            pltpu.VMEM((128, D), jnp.float32),
            pltpu.VMEM((L,), jnp.int32),
            pltpu.VMEM_SHARED((N, D), jnp.float32),
            pltpu.VMEM_SHARED((N + L,), jnp.float32),
            pltpu.SemaphoreType.DMA,
            pltpu.SemaphoreType.DMA,
            pltpu.SemaphoreType.DMA,
        ],
    )(edge_row, edge_col, edge_vals, m, xw, nid_vec)


def _tc_fin_body(h1p_ref, cp_ref, wm2_ref, out_ref):
    h1 = jnp.maximum(h1p_ref[0] + h1p_ref[1], 0.0)
    coeff = (cp_ref[0] + cp_ref[1])[None, :]
    t = jnp.dot(coeff, h1, preferred_element_type=jnp.float32, precision=lax.Precision.HIGHEST)
    pred = jnp.dot(t, wm2_ref[...], preferred_element_type=jnp.float32, precision=lax.Precision.HIGHEST)
    m = jnp.max(pred, axis=1, keepdims=True)
    ex = jnp.exp(pred - m)
    out_ref[...] = ex / jnp.sum(ex, axis=1, keepdims=True)


def _tc_fin(h1p, cp, wm2):
    return pl.pallas_call(
        _tc_fin_body,
        out_shape=jax.ShapeDtypeStruct((1, C), jnp.float32),
    )(h1p, cp, wm2)


def kernel(x, embed, edge_vals, W1e, b1e, W2e, b2e, Wm1, Wm2,
           edge_row, edge_col, nodeid, tmp):
    w1a = W1e[:D]
    w1b = W1e[D:2 * D]
    w1c = W1e[2 * D:]
    b1 = b1e.reshape(1, H)
    embn = lax.dynamic_slice_in_dim(embed, nodeid, 1, axis=0)
    w2b = jnp.broadcast_to(W2e.reshape(H, 1), (H, L))
    b2b = jnp.broadcast_to(b2e.reshape(1), (L,))
    nid_vec = jnp.full((L,), nodeid, jnp.int32)

    pq, xw = _tc_pre(embed, x, w1a, w1b, w1c, b1, embn, Wm1)
    vals = _sc_a(pq, edge_row, edge_col, w2b, b2b)
    m = _sc_b(edge_row, edge_col, vals)
    h1p, cp = _sc_c(edge_row, edge_col, edge_vals, m, xw, nid_vec)
    res = _tc_fin(h1p, cp, Wm2)
    return res.reshape(C)


# speedup vs baseline: 2.0958x; 2.0958x over previous
import functools
import jax
import jax.numpy as jnp
from jax import lax
from jax.experimental import pallas as pl
from jax.experimental.pallas import tpu as pltpu, tpu_sc as plsc

N = 4096
E = 131072
D = 128
H = 64
C = 16

NC, NS, L = 2, 16, 16
NW = NC * NS

CH_ROWS = 256
CH_W = CH_ROWS * N
N_SWEEPS = N // (CH_ROWS * NC)
DUMMY_B = CH_W
DUMMY_C = N

_mesh = plsc.VectorSubcoreMesh(core_axis_name="c", subcore_axis_name="s")
_sc_params = pltpu.CompilerParams(needs_layout_passes=False)


def _sigmoid(t):
    return 1.0 / (1.0 + jnp.exp(-t))


def _tc_pre_body(emb_ref, x_ref, w1a_ref, w1b_ref, w1c_ref, b1_ref,
                 embn_ref, wm1_ref, p_ref, q_ref, xw_ref):
    kv = jnp.dot(embn_ref[...], w1c_ref[...],
                 preferred_element_type=jnp.float32, precision=lax.Precision.HIGHEST) + b1_ref[...]
    p = jnp.dot(emb_ref[...], w1a_ref[...],
                preferred_element_type=jnp.float32, precision=lax.Precision.HIGHEST) + kv
    q = jnp.dot(emb_ref[...], w1b_ref[...],
                preferred_element_type=jnp.float32, precision=lax.Precision.HIGHEST)
    p_ref[...] = jnp.concatenate([p, q], axis=1)
    q_ref[...] = jnp.concatenate([q, p], axis=1)
    xw_ref[...] = jnp.dot(x_ref[...], wm1_ref[...],
                          preferred_element_type=jnp.float32, precision=lax.Precision.HIGHEST)


def _tc_pre(embed, x, w1a, w1b, w1c, b1, embn, wm1):
    return pl.pallas_call(
        _tc_pre_body,
        out_shape=[
            jax.ShapeDtypeStruct((N, 2 * H), jnp.float32),
            jax.ShapeDtypeStruct((N, 2 * H), jnp.float32),
            jax.ShapeDtypeStruct((N, D), jnp.float32),
        ],
    )(embed, x, w1a, w1b, w1c, b1, embn, wm1)


EW_A = E // NW
KA = 256


def _sc_a_body(pq_hbm, qp_hbm, row_hbm, col_hbm, w2b_hbm, b2b_hbm, vals_hbm,
               rbuf, cbuf, sg, vbuf, w2b, b2b, semr, semg, sema):
    cid = lax.axis_index("c")
    sid = lax.axis_index("s")
    wid = sid * NC + cid
    pltpu.sync_copy(w2b_hbm, w2b)
    pltpu.sync_copy(b2b_hbm, b2b)
    lanes = lax.iota(jnp.int32, L)

    def chunk(ch, _):
        base = wid * EW_A + ch * KA
        cl1 = pltpu.async_copy(row_hbm.at[pl.ds(base, KA)], rbuf, semr)
        cl2 = pltpu.async_copy(col_hbm.at[pl.ds(base, KA)], cbuf, semr)
        cl1.wait()
        cl2.wait()
        gps = []
        for j in range(KA // 128):
            gps.append(pltpu.async_copy(
                pq_hbm.at[rbuf.at[pl.ds(j * 128, 128)]],
                sg.at[pl.ds(j * 128, 128)], semg))
        aps = []
        for j in range(KA // 128):
            gps[j].wait()
            aps.append(pltpu.async_copy(
                qp_hbm.at[cbuf.at[pl.ds(j * 128, 128)]],
                sg.at[pl.ds(j * 128, 128)], sema, add=True))
        for cp in aps:
            cp.wait()

        b2 = b2b[pl.ds(0, L)]

        def block(b, _):
            ev = b * L + lanes
            acc = jnp.zeros((L,), jnp.float32)
            for d in range(H):
                dv = jnp.full((L,), d, jnp.int32)
                sv = plsc.load_gather(sg, [ev, dv])
                acc = acc + jnp.maximum(sv, 0.0) * w2b[d, pl.ds(0, L)]
            vbuf[pl.ds(b * L, L)] = _sigmoid(acc + b2)
            return _
        lax.fori_loop(0, KA // L, block, 0)
        pltpu.sync_copy(vbuf, vals_hbm.at[pl.ds(base, KA)])
        return _
    lax.fori_loop(0, EW_A // KA, chunk, 0)


def _sc_a(pq, qp, edge_row, edge_col, w2b, b2b):
    return pl.kernel(
        _sc_a_body,
        out_type=jax.ShapeDtypeStruct((E,), jnp.float32),
        mesh=_mesh,
        compiler_params=_sc_params,
        scratch_types=[
            pltpu.VMEM((KA,), jnp.int32),
            pltpu.VMEM((KA,), jnp.int32),
            pltpu.VMEM((KA, 2 * H), jnp.float32),
            pltpu.VMEM((KA,), jnp.float32),
            pltpu.VMEM((H, L), jnp.float32),
            pltpu.VMEM((L,), jnp.float32),
            pltpu.SemaphoreType.DMA,
            pltpu.SemaphoreType.DMA,
            pltpu.SemaphoreType.DMA,
        ],
    )(pq, qp, edge_row, edge_col, w2b, b2b)


EW_B = E // NS
KB = 1024


def _sc_b_body(row_hbm, col_hbm, vals_hbm, m_hbm,
               rbuf, cbuf, vbuf, sidx, sval, zbuf, acc, sem, semr):
    cid = lax.axis_index("c")
    sid = lax.axis_index("s")
    zv = jnp.zeros((L,), jnp.float32)

    def zb(i, _):
        zbuf[pl.ds(i * L, L)] = zv
        return _
    lax.fori_loop(0, 16384 // L, zb, 0)
    lanes = lax.iota(jnp.int32, L)

    def sweep(s, _):
        base_row = s * (CH_ROWS * NC) + cid * CH_ROWS
        for j in range(4):
            pltpu.sync_copy(zbuf, acc.at[pl.ds(sid * 65536 + j * 16384, 16384)])
        plsc.subcore_barrier()

        def chunk(ch, off):
            ebase = sid * EW_B + ch * KB
            c1 = pltpu.async_copy(row_hbm.at[pl.ds(ebase, KB)], rbuf, semr)
            c2 = pltpu.async_copy(col_hbm.at[pl.ds(ebase, KB)], cbuf, semr)
            c3 = pltpu.async_copy(vals_hbm.at[pl.ds(ebase, KB)], vbuf, semr)
            c1.wait()
            c2.wait()
            c3.wait()

            def block(b, off):
                r16 = rbuf[pl.ds(b * L, L)]
                c16 = cbuf[pl.ds(b * L, L)]
                v16 = vbuf[pl.ds(b * L, L)]
                lidx = (r16 - base_row) * N + c16
                m = (r16 >= base_row) & (r16 < base_row + CH_ROWS)
                plsc.store_compressed(sidx.at[pl.ds(off, L)], lidx, mask=m)
                plsc.store_compressed(sval.at[pl.ds(off, L)], v16, mask=m)
                return off + jnp.max(plsc.all_reduce_population_count(m))
            return lax.fori_loop(0, KB // L, block, off)
        tot = lax.fori_loop(0, EW_B // KB, chunk, 0)

        ones = jnp.full((L,), True)
        for k in range(8):
            plsc.store_compressed(sidx.at[pl.ds(tot + k * L, L)],
                                  jnp.full((L,), DUMMY_B, jnp.int32), mask=ones)
            plsc.store_compressed(sval.at[pl.ds(tot + k * L, L)], zv, mask=ones)

        def srow(j, _):
            pltpu.sync_copy(sval.at[pl.ds(j * 128, 128)],
                            acc.at[sidx.at[pl.ds(j * 128, 128)]], add=True)
            return _
        lax.fori_loop(0, (tot + 127) // 128, srow, 0)
        plsc.subcore_barrier()
        pltpu.sync_copy(
            acc.at[pl.ds(sid * 65536, 65536)],
            m_hbm.at[pl.ds(base_row * N + sid * 65536, 65536)])
        plsc.subcore_barrier()
        return _
    lax.fori_loop(0, N_SWEEPS, sweep, 0)


def _sc_b(edge_row, edge_col, vals):
    return pl.kernel(
        _sc_b_body,
        out_type=jax.ShapeDtypeStruct((N * N,), jnp.float32),
        mesh=_mesh,
        compiler_params=_sc_params,
        scratch_types=[
            pltpu.VMEM((KB,), jnp.int32),
            pltpu.VMEM((KB,), jnp.int32),
            pltpu.VMEM((KB,), jnp.float32),
            pltpu.VMEM((EW_B + 128,), jnp.int32),
            pltpu.VMEM((EW_B + 128,), jnp.float32),
            pltpu.VMEM((16384,), jnp.float32),
            pltpu.VMEM_SHARED((CH_W + 64,), jnp.float32),
            pltpu.SemaphoreType.DMA,
            pltpu.SemaphoreType.DMA,
        ],
    )(edge_row, edge_col, vals)


EW_C = E // NW
KC = 256


def _sc_c_body(row_hbm, col_hbm, ev_hbm, m_hbm, xw_hbm, nid_hbm,
               h1p_hbm, cp_hbm,
               rbuf, cbuf, evbuf, fridx, cidx, frw, wbuf,
               xwg, zbuf2, nbuf, hacc, cacc,
               semr, sem, sem3):
    cid = lax.axis_index("c")
    sid = lax.axis_index("s")
    zv = jnp.zeros((L,), jnp.float32)

    def zb(i, _):
        zbuf2[i // 8, pl.ds((i % 8) * L, L)] = zv
        return _
    lax.fori_loop(0, 128 * D // L, zb, 0)
    for j in range(2):
        pltpu.sync_copy(zbuf2, hacc.at[pl.ds(sid * 256 + j * 128, 128)])

    @pl.when(sid == 0)
    def _():
        for j in range(N // 128):
            pltpu.sync_copy(zbuf2.at[0], cacc.at[pl.ds(j * 128, 128)])
    pltpu.sync_copy(nid_hbm, nbuf)
    plsc.subcore_barrier()

    lanes = lax.iota(jnp.int32, L)

    def chunk(ch, _):
        gwid = cid * NS + sid
        ebase = gwid * EW_C + ch * KC
        c1 = pltpu.async_copy(row_hbm.at[pl.ds(ebase, KC)], rbuf, semr)
        c2 = pltpu.async_copy(col_hbm.at[pl.ds(ebase, KC)], cbuf, semr)
        c3 = pltpu.async_copy(ev_hbm.at[pl.ds(ebase, KC)], evbuf, semr)
        c1.wait()
        c2.wait()
        c3.wait()
        nid = nbuf[pl.ds(0, L)]

        def block(b, _):
            r16 = rbuf[pl.ds(b * L, L)]
            c16 = cbuf[pl.ds(b * L, L)]
            fridx[pl.ds(b * L, L)] = r16 * N + c16
            fridx[pl.ds(KC + b * L, L)] = c16 * N + r16
            cidx[pl.ds(b * L, L)] = jnp.where(r16 == nid, c16, DUMMY_C)
            return _
        lax.fori_loop(0, KC // L, block, 0)
        gps = []
        for j in range(2 * KC // 128):
            gps.append(pltpu.async_copy(
                m_hbm.at[fridx.at[pl.ds(j * 128, 128)]],
                frw.at[pl.ds(j * 128, 128)], sem))
        for j in range(KC // 128):
            gps.append(pltpu.async_copy(
                xw_hbm.at[cbuf.at[pl.ds(j * 128, 128)]],
                xwg.at[pl.ds(j * 128, 128)], sem3))
        for cp in gps:
            cp.wait()

        def wblock(b, _):
            r16 = rbuf[pl.ds(b * L, L)]
            c16 = cbuf[pl.ds(b * L, L)]
            e16 = evbuf[pl.ds(b * L, L)]
            f16 = frw[pl.ds(b * L, L)]
            v16 = frw[pl.ds(KC + b * L, L)]
            w = e16 * 0.5 * (f16 + v16)
            w = jnp.where(r16 == c16, 0.0, w)
            wbuf[pl.ds(b * L, L)] = w
            return _
        lax.fori_loop(0, KC // L, wblock, 0)

        def scale(eb, _):
            ev = eb * L + lanes
            w16 = wbuf[pl.ds(eb * L, L)]
            for d in range(D):
                dv = jnp.full((L,), d, jnp.int32)
                v = plsc.load_gather(xwg, [ev, dv])
                plsc.store_scatter(xwg, [ev, dv], v * w16)
            return _
        lax.fori_loop(0, KC // L, scale, 0)

        for j in range(KC // 128):
            pltpu.sync_copy(xwg.at[pl.ds(j * 128, 128)],
                            hacc.at[rbuf.at[pl.ds(j * 128, 128)]], add=True)
        pltpu.sync_copy(wbuf, cacc.at[cidx], add=True)
        return _
    lax.fori_loop(0, EW_C // KC, chunk, 0)
    plsc.subcore_barrier()
    for j in range(2):
        pltpu.sync_copy(hacc.at[pl.ds(sid * 256 + j * 128, 128)],
                        h1p_hbm.at[cid, pl.ds(sid * 256 + j * 128, 128)])

    @pl.when(sid == 0)
    def _():
        pltpu.sync_copy(cacc.at[pl.ds(0, N)], cp_hbm.at[cid])


def _sc_c(edge_row, edge_col, edge_vals, m, xw, nid_vec):
    return pl.kernel(
        _sc_c_body,
        out_type=(
            jax.ShapeDtypeStruct((NC, N, D), jnp.float32),
            jax.ShapeDtypeStruct((NC, N), jnp.float32),
        ),
        mesh=_mesh,
        compiler_params=_sc_params,
        scratch_types=[
            pltpu.VMEM((KC,), jnp.int32),
            pltpu.VMEM((KC,), jnp.int32),
            pltpu.VMEM((KC,), jnp.float32),
            pltpu.VMEM((2 * KC,), jnp.int32),
            pltpu.VMEM((KC,), jnp.int32),
            pltpu.VMEM((2 * KC,), jnp.float32),
            pltpu.VMEM((KC,), jnp.float32),
            pltpu.VMEM((KC, D), jnp.float32),
            pltpu.VMEM((128, D), jnp.float32),
            pltpu.VMEM((L,), jnp.int32),
            pltpu.VMEM_SHARED((N, D), jnp.float32),
            pltpu.VMEM_SHARED((N + L,), jnp.float32),
            pltpu.SemaphoreType.DMA,
            pltpu.SemaphoreType.DMA,
            pltpu.SemaphoreType.DMA,
        ],
    )(edge_row, edge_col, edge_vals, m, xw, nid_vec)


def _tc_fin_body(h1p_ref, cp_ref, wm2_ref, out_ref):
    h1 = jnp.maximum(h1p_ref[0] + h1p_ref[1], 0.0)
    coeff = (cp_ref[0] + cp_ref[1])[None, :]
    t = jnp.dot(coeff, h1, preferred_element_type=jnp.float32, precision=lax.Precision.HIGHEST)
    pred = jnp.dot(t, wm2_ref[...], preferred_element_type=jnp.float32, precision=lax.Precision.HIGHEST)
    m = jnp.max(pred, axis=1, keepdims=True)
    ex = jnp.exp(pred - m)
    out_ref[...] = ex / jnp.sum(ex, axis=1, keepdims=True)


def _tc_fin(h1p, cp, wm2):
    return pl.pallas_call(
        _tc_fin_body,
        out_shape=jax.ShapeDtypeStruct((1, C), jnp.float32),
    )(h1p, cp, wm2)


def kernel(x, embed, edge_vals, W1e, b1e, W2e, b2e, Wm1, Wm2,
           edge_row, edge_col, nodeid, tmp):
    w1a = W1e[:D]
    w1b = W1e[D:2 * D]
    w1c = W1e[2 * D:]
    b1 = b1e.reshape(1, H)
    embn = lax.dynamic_slice_in_dim(embed, nodeid, 1, axis=0)
    w2b = jnp.broadcast_to(W2e.reshape(H, 1), (H, L))
    b2b = jnp.broadcast_to(b2e.reshape(1), (L,))
    nid_vec = jnp.full((L,), nodeid, jnp.int32)

    pq, qp, xw = _tc_pre(embed, x, w1a, w1b, w1c, b1, embn, Wm1)
    vals = _sc_a(pq, qp, edge_row, edge_col, w2b, b2b)
    m = _sc_b(edge_row, edge_col, vals)
    h1p, cp = _sc_c(edge_row, edge_col, edge_vals, m, xw, nid_vec)
    res = _tc_fin(h1p, cp, Wm2)
    return res.reshape(C)


# speedup vs baseline: 2.1652x; 1.0331x over previous
import functools
import jax
import jax.numpy as jnp
from jax import lax
from jax.experimental import pallas as pl
from jax.experimental.pallas import tpu as pltpu, tpu_sc as plsc

N = 4096
E = 131072
D = 128
H = 64
C = 16

NC, NS, L = 2, 16, 16
NW = NC * NS

CH_ROWS = 256
CH_W = CH_ROWS * N
N_SWEEPS = N // (CH_ROWS * NC)
DUMMY_B = CH_W
DUMMY_C = N

_mesh = plsc.VectorSubcoreMesh(core_axis_name="c", subcore_axis_name="s")
_sc_params = pltpu.CompilerParams(needs_layout_passes=False)


def _sigmoid(t):
    return 1.0 / (1.0 + jnp.exp(-t))


def _tc_pre_body(emb_ref, x_ref, w1a_ref, w1b_ref, w1c_ref, b1_ref,
                 embn_ref, wm1_ref, p_ref, q_ref, xw_ref):
    kv = jnp.dot(embn_ref[...], w1c_ref[...],
                 preferred_element_type=jnp.float32, precision=lax.Precision.HIGHEST) + b1_ref[...]
    p = jnp.dot(emb_ref[...], w1a_ref[...],
                preferred_element_type=jnp.float32, precision=lax.Precision.HIGHEST) + kv
    q = jnp.dot(emb_ref[...], w1b_ref[...],
                preferred_element_type=jnp.float32, precision=lax.Precision.HIGHEST)
    p_ref[...] = jnp.concatenate([p, q], axis=1)
    q_ref[...] = jnp.concatenate([q, p], axis=1)
    xw_ref[...] = jnp.dot(x_ref[...], wm1_ref[...],
                          preferred_element_type=jnp.float32, precision=lax.Precision.HIGHEST)


def _tc_pre(embed, x, w1a, w1b, w1c, b1, embn, wm1):
    return pl.pallas_call(
        _tc_pre_body,
        out_shape=[
            jax.ShapeDtypeStruct((N, 2 * H), jnp.float32),
            jax.ShapeDtypeStruct((N, 2 * H), jnp.float32),
            jax.ShapeDtypeStruct((N, D), jnp.float32),
        ],
    )(embed, x, w1a, w1b, w1c, b1, embn, wm1)


EW_A = E // NW
KA = 256


def _sc_a_body(pq_hbm, qp_hbm, row_hbm, col_hbm, w2b_hbm, b2b_hbm, vals_hbm,
               rbuf, cbuf, sg, vbuf, w2b, b2b, semr, semg, sema):
    cid = lax.axis_index("c")
    sid = lax.axis_index("s")
    wid = sid * NC + cid
    pltpu.sync_copy(w2b_hbm, w2b)
    pltpu.sync_copy(b2b_hbm, b2b)
    lanes = lax.iota(jnp.int32, L)

    def chunk(ch, _):
        base = wid * EW_A + ch * KA
        cl1 = pltpu.async_copy(row_hbm.at[pl.ds(base, KA)], rbuf, semr)
        cl2 = pltpu.async_copy(col_hbm.at[pl.ds(base, KA)], cbuf, semr)
        cl1.wait()
        cl2.wait()
        gps = []
        for j in range(KA // 128):
            gps.append(pltpu.async_copy(
                pq_hbm.at[rbuf.at[pl.ds(j * 128, 128)]],
                sg.at[pl.ds(j * 128, 128)], semg))
        aps = []
        for j in range(KA // 128):
            gps[j].wait()
            aps.append(pltpu.async_copy(
                qp_hbm.at[cbuf.at[pl.ds(j * 128, 128)]],
                sg.at[pl.ds(j * 128, 128)], sema, add=True))
        for cp in aps:
            cp.wait()

        b2 = b2b[pl.ds(0, L)]

        def block(b, _):
            ev = b * L + lanes
            acc = jnp.zeros((L,), jnp.float32)
            for d in range(H):
                dv = jnp.full((L,), d, jnp.int32)
                sv = plsc.load_gather(sg, [ev, dv])
                acc = acc + jnp.maximum(sv, 0.0) * w2b[d, pl.ds(0, L)]
            vbuf[pl.ds(b * L, L)] = _sigmoid(acc + b2)
            return _
        lax.fori_loop(0, KA // L, block, 0)
        pltpu.sync_copy(vbuf, vals_hbm.at[pl.ds(base, KA)])
        return _
    lax.fori_loop(0, EW_A // KA, chunk, 0)


def _sc_a(pq, qp, edge_row, edge_col, w2b, b2b):
    return pl.kernel(
        _sc_a_body,
        out_type=jax.ShapeDtypeStruct((E,), jnp.float32),
        mesh=_mesh,
        compiler_params=_sc_params,
        scratch_types=[
            pltpu.VMEM((KA,), jnp.int32),
            pltpu.VMEM((KA,), jnp.int32),
            pltpu.VMEM((KA, 2 * H), jnp.float32),
            pltpu.VMEM((KA,), jnp.float32),
            pltpu.VMEM((H, L), jnp.float32),
            pltpu.VMEM((L,), jnp.float32),
            pltpu.SemaphoreType.DMA,
            pltpu.SemaphoreType.DMA,
            pltpu.SemaphoreType.DMA,
        ],
    )(pq, qp, edge_row, edge_col, w2b, b2b)


EW_B = E // NS
KB = 1024


def _sc_b_body(row_hbm, col_hbm, vals_hbm, m_hbm,
               rbuf, cbuf, vbuf, sidx, sval, zbuf, acc, sem, semr):
    cid = lax.axis_index("c")
    sid = lax.axis_index("s")
    zv = jnp.zeros((L,), jnp.float32)

    def zb(i, _):
        zbuf[pl.ds(i * L, L)] = zv
        return _
    lax.fori_loop(0, 16384 // L, zb, 0)
    lanes = lax.iota(jnp.int32, L)

    def sweep(s, _):
        base_row = s * (CH_ROWS * NC) + cid * CH_ROWS
        for j in range(4):
            pltpu.sync_copy(zbuf, acc.at[pl.ds(sid * 65536 + j * 16384, 16384)])
        plsc.subcore_barrier()

        def chunk(ch, off):
            ebase = sid * EW_B + ch * KB
            c1 = pltpu.async_copy(row_hbm.at[pl.ds(ebase, KB)], rbuf, semr)
            c2 = pltpu.async_copy(col_hbm.at[pl.ds(ebase, KB)], cbuf, semr)
            c3 = pltpu.async_copy(vals_hbm.at[pl.ds(ebase, KB)], vbuf, semr)
            c1.wait()
            c2.wait()
            c3.wait()

            def block(b, off):
                r16 = rbuf[pl.ds(b * L, L)]
                c16 = cbuf[pl.ds(b * L, L)]
                v16 = vbuf[pl.ds(b * L, L)]
                lidx = (r16 - base_row) * N + c16
                m = (r16 >= base_row) & (r16 < base_row + CH_ROWS)
                plsc.store_compressed(sidx.at[pl.ds(off, L)], lidx, mask=m)
                plsc.store_compressed(sval.at[pl.ds(off, L)], v16, mask=m)
                return off + jnp.max(plsc.all_reduce_population_count(m))
            return lax.fori_loop(0, KB // L, block, off)
        tot = lax.fori_loop(0, EW_B // KB, chunk, 0)

        ones = jnp.full((L,), True)
        for k in range(8):
            plsc.store_compressed(sidx.at[pl.ds(tot + k * L, L)],
                                  jnp.full((L,), DUMMY_B, jnp.int32), mask=ones)
            plsc.store_compressed(sval.at[pl.ds(tot + k * L, L)], zv, mask=ones)

        def srow(j, _):
            pltpu.sync_copy(sval.at[pl.ds(j * 128, 128)],
                            acc.at[sidx.at[pl.ds(j * 128, 128)]], add=True)
            return _
        lax.fori_loop(0, (tot + 127) // 128, srow, 0)
        plsc.subcore_barrier()
        pltpu.sync_copy(
            acc.at[pl.ds(sid * 65536, 65536)],
            m_hbm.at[pl.ds(base_row * N + sid * 65536, 65536)])
        plsc.subcore_barrier()
        return _
    lax.fori_loop(0, N_SWEEPS, sweep, 0)


def _sc_b(edge_row, edge_col, vals):
    return pl.kernel(
        _sc_b_body,
        out_type=jax.ShapeDtypeStruct((N * N,), jnp.float32),
        mesh=_mesh,
        compiler_params=_sc_params,
        scratch_types=[
            pltpu.VMEM((KB,), jnp.int32),
            pltpu.VMEM((KB,), jnp.int32),
            pltpu.VMEM((KB,), jnp.float32),
            pltpu.VMEM((EW_B + 128,), jnp.int32),
            pltpu.VMEM((EW_B + 128,), jnp.float32),
            pltpu.VMEM((16384,), jnp.float32),
            pltpu.VMEM_SHARED((CH_W + 64,), jnp.float32),
            pltpu.SemaphoreType.DMA,
            pltpu.SemaphoreType.DMA,
        ],
    )(edge_row, edge_col, vals)


EW_C = E // NW
KC = 256


def _sc_c_body(row_hbm, col_hbm, ev_hbm, m_hbm, xw_hbm, nid_hbm,
               h1p_hbm, cp_hbm,
               rbuf0, rbuf1, cbuf0, cbuf1, evbuf0, evbuf1,
               fridx0, fridx1, sidx0, sidx1, cidx0, cidx1,
               frw0, frw1, wbuf0, wbuf1, xwg0, xwg1,
               zbuf2, nbuf, hacc, cacc,
               semL0, semL1, semG, semX, semS):
    cid = lax.axis_index("c")
    sid = lax.axis_index("s")
    zv = jnp.zeros((L,), jnp.float32)

    def zb(i, _):
        zbuf2[i // 8, pl.ds((i % 8) * L, L)] = zv
        return _
    lax.fori_loop(0, 128 * D // L, zb, 0)
    for j in range(2):
        pltpu.sync_copy(zbuf2, hacc.at[pl.ds(sid * 256 + j * 128, 128)])

    @pl.when(sid == 0)
    def _():
        for j in range(N // 128):
            pltpu.sync_copy(zbuf2.at[0], cacc.at[pl.ds(j * 128, 128)])
    pltpu.sync_copy(nid_hbm, nbuf)
    plsc.subcore_barrier()

    lanes = lax.iota(jnp.int32, L)
    nid = nbuf[pl.ds(0, L)]
    gwid = cid * NS + sid
    NCH = EW_C // KC

    rb = [rbuf0, rbuf1]
    cb = [cbuf0, cbuf1]
    eb = [evbuf0, evbuf1]
    fx = [fridx0, fridx1]
    sx = [sidx0, sidx1]
    cx = [cidx0, cidx1]
    fr = [frw0, frw1]
    wb = [wbuf0, wbuf1]
    xg = [xwg0, xwg1]
    semL = [semL0, semL1]

    def fire_lin(i):
        s = i % 2
        base = gwid * EW_C + i * KC
        return [
            pltpu.async_copy(row_hbm.at[pl.ds(base, KC)], rb[s], semL[s]),
            pltpu.async_copy(col_hbm.at[pl.ds(base, KC)], cb[s], semL[s]),
            pltpu.async_copy(ev_hbm.at[pl.ds(base, KC)], eb[s], semL[s]),
        ]

    def idx_compute(i):
        s = i % 2

        def block(b, _):
            r16 = rb[s][pl.ds(b * L, L)]
            c16 = cb[s][pl.ds(b * L, L)]
            fx[s][pl.ds(b * L, L)] = r16 * N + c16
            fx[s][pl.ds(KC + b * L, L)] = c16 * N + r16
            sx[s][pl.ds(b * L, L)] = r16
            cx[s][pl.ds(b * L, L)] = jnp.where(r16 == nid, c16, DUMMY_C)
            return _
        lax.fori_loop(0, KC // L, block, 0)

    def fire_gath(i):
        s = i % 2
        cps = []
        for j in range(2 * KC // 128):
            cps.append(pltpu.async_copy(
                m_hbm.at[fx[s].at[pl.ds(j * 128, 128)]],
                fr[s].at[pl.ds(j * 128, 128)], semG))
        for j in range(KC // 128):
            cps.append(pltpu.async_copy(
                xw_hbm.at[cb[s].at[pl.ds(j * 128, 128)]],
                xg[s].at[pl.ds(j * 128, 128)], semX))
        return cps

    def compute(i):
        s = i % 2

        def wblock(b, _):
            r16 = rb[s][pl.ds(b * L, L)]
            c16 = cb[s][pl.ds(b * L, L)]
            e16 = eb[s][pl.ds(b * L, L)]
            f16 = fr[s][pl.ds(b * L, L)]
            v16 = fr[s][pl.ds(KC + b * L, L)]
            w = e16 * 0.5 * (f16 + v16)
            w = jnp.where(r16 == c16, 0.0, w)
            wb[s][pl.ds(b * L, L)] = w
            return _
        lax.fori_loop(0, KC // L, wblock, 0)

        def scale(eb_, _):
            ev = eb_ * L + lanes
            w16 = wb[s][pl.ds(eb_ * L, L)]

            def dloop(d, _):
                dv = jnp.full((L,), 0, jnp.int32) + d
                v = plsc.load_gather(xg[s], [ev, dv])
                plsc.store_scatter(xg[s], [ev, dv], v * w16)
                return _
            lax.fori_loop(0, D, dloop, 0)
            return _
        lax.fori_loop(0, KC // L, scale, 0)

    def fire_scat(i):
        s = i % 2
        cps = []
        for j in range(KC // 128):
            cps.append(pltpu.async_copy(
                xg[s].at[pl.ds(j * 128, 128)],
                hacc.at[sx[s].at[pl.ds(j * 128, 128)]], semS, add=True))
        cps.append(pltpu.async_copy(wb[s], cacc.at[cx[s]], semS, add=True))
        return cps

    lin = {0: fire_lin(0)}
    for cp in lin[0]:
        cp.wait()
    idx_compute(0)
    gath = {0: fire_gath(0)}
    lin[1] = fire_lin(1)
    scat = {}
    for i in range(NCH):
        for cp in gath[i]:
            cp.wait()
        if i >= 1:
            for cp in scat[i - 1]:
                cp.wait()
        if i + 1 < NCH:
            for cp in lin[i + 1]:
                cp.wait()
            idx_compute(i + 1)
            gath[i + 1] = fire_gath(i + 1)
        compute(i)
        scat[i] = fire_scat(i)
        if i + 2 < NCH:
            lin[i + 2] = fire_lin(i + 2)
    for cp in scat[NCH - 1]:
        cp.wait()
    plsc.subcore_barrier()
    for j in range(2):
        pltpu.sync_copy(hacc.at[pl.ds(sid * 256 + j * 128, 128)],
                        h1p_hbm.at[cid, pl.ds(sid * 256 + j * 128, 128)])

    @pl.when(sid == 0)
    def _():
        pltpu.sync_copy(cacc.at[pl.ds(0, N)], cp_hbm.at[cid])


def _sc_c(edge_row, edge_col, edge_vals, m, xw, nid_vec):
    return pl.kernel(
        _sc_c_body,
        out_type=(
            jax.ShapeDtypeStruct((NC, N, D), jnp.float32),
            jax.ShapeDtypeStruct((NC, N), jnp.float32),
        ),
        mesh=_mesh,
        compiler_params=_sc_params,
        scratch_types=[
            pltpu.VMEM((KC,), jnp.int32),
            pltpu.VMEM((KC,), jnp.int32),
            pltpu.VMEM((KC,), jnp.int32),
            pltpu.VMEM((KC,), jnp.int32),
            pltpu.VMEM((KC,), jnp.float32),
            pltpu.VMEM((KC,), jnp.float32),
            pltpu.VMEM((2 * KC,), jnp.int32),
            pltpu.VMEM((2 * KC,), jnp.int32),
            pltpu.VMEM((KC,), jnp.int32),
            pltpu.VMEM((KC,), jnp.int32),
            pltpu.VMEM((KC,), jnp.int32),
            pltpu.VMEM((KC,), jnp.int32),
            pltpu.VMEM((2 * KC,), jnp.float32),
            pltpu.VMEM((2 * KC,), jnp.float32),
            pltpu.VMEM((KC,), jnp.float32),
            pltpu.VMEM((KC,), jnp.float32),
            pltpu.VMEM((KC, D), jnp.float32),
            pltpu.VMEM((KC, D), jnp.float32),
            pltpu.VMEM((128, D), jnp.float32),
            pltpu.VMEM((L,), jnp.int32),
            pltpu.VMEM_SHARED((N, D), jnp.float32),
            pltpu.VMEM_SHARED((N + L,), jnp.float32),
            pltpu.SemaphoreType.DMA,
            pltpu.SemaphoreType.DMA,
            pltpu.SemaphoreType.DMA,
            pltpu.SemaphoreType.DMA,
            pltpu.SemaphoreType.DMA,
        ],
    )(edge_row, edge_col, edge_vals, m, xw, nid_vec)


def _tc_fin_body(h1p_ref, cp_ref, wm2_ref, out_ref):
    h1 = jnp.maximum(h1p_ref[0] + h1p_ref[1], 0.0)
    coeff = (cp_ref[0] + cp_ref[1])[None, :]
    t = jnp.dot(coeff, h1, preferred_element_type=jnp.float32, precision=lax.Precision.HIGHEST)
    pred = jnp.dot(t, wm2_ref[...], preferred_element_type=jnp.float32, precision=lax.Precision.HIGHEST)
    m = jnp.max(pred, axis=1, keepdims=True)
    ex = jnp.exp(pred - m)
    out_ref[...] = ex / jnp.sum(ex, axis=1, keepdims=True)


def _tc_fin(h1p, cp, wm2):
    return pl.pallas_call(
        _tc_fin_body,
        out_shape=jax.ShapeDtypeStruct((1, C), jnp.float32),
    )(h1p, cp, wm2)


def kernel(x, embed, edge_vals, W1e, b1e, W2e, b2e, Wm1, Wm2,
           edge_row, edge_col, nodeid, tmp):
    w1a = W1e[:D]
    w1b = W1e[D:2 * D]
    w1c = W1e[2 * D:]
    b1 = b1e.reshape(1, H)
    embn = lax.dynamic_slice_in_dim(embed, nodeid, 1, axis=0)
    w2b = jnp.broadcast_to(W2e.reshape(H, 1), (H, L))
    b2b = jnp.broadcast_to(b2e.reshape(1), (L,))
    nid_vec = jnp.full((L,), nodeid, jnp.int32)

    pq, qp, xw = _tc_pre(embed, x, w1a, w1b, w1c, b1, embn, Wm1)
    vals = _sc_a(pq, qp, edge_row, edge_col, w2b, b2b)
    m = _sc_b(edge_row, edge_col, vals)
    h1p, cp = _sc_c(edge_row, edge_col, edge_vals, m, xw, nid_vec)
    res = _tc_fin(h1p, cp, Wm2)
    return res.reshape(C)
